# baseline (speedup 1.0000x reference)
import functools

import jax
import jax.numpy as jnp
from jax import lax
from jax.experimental import pallas as pl
from jax.experimental.pallas import tpu as pltpu
from jax.experimental.pallas import tpu_sc as plsc

VOCAB = 49408
H = 512
S = 16
BATCH = 4096
N = BATCH * S
L = 16
NC, NS = 2, 16
NW = NC * NS
BPW = N // NW
C = 16
NCHUNK = BPW // C
NBUF = 8
G = 4

_mesh = plsc.VectorSubcoreMesh(core_axis_name="c", subcore_axis_name="s")


@functools.partial(
    pl.kernel,
    out_type=jax.ShapeDtypeStruct((N, H), jnp.float32),
    mesh=_mesh,
    scratch_types=[
        pltpu.VMEM((NCHUNK, C), jnp.int32),
        pltpu.VMEM((S, H), jnp.float32),
    ] + [pltpu.VMEM((C, H), jnp.float32) for _ in range(NBUF)]
      + [pltpu.SemaphoreType.DMA for _ in range(2 * NBUF)],
)
def _emb(ids_hbm, tok_hbm, pos_hbm, out_hbm, idx_v, pos_v, *bs):
    bufs = bs[:NBUF]
    gsem = bs[NBUF:2 * NBUF]
    ssem = bs[2 * NBUF:]
    wid = lax.axis_index("s") * NC + lax.axis_index("c")
    base = wid * BPW
    pltpu.sync_copy(ids_hbm.at[wid], idx_v)
    pltpu.sync_copy(pos_hbm, pos_v)

    def add_pos(rows):
        def jbody(j, c):
            off = j * L
            ps = [pos_v[s, pl.ds(off, L)] for s in range(S)]
            for g in range(C // S):
                for s in range(S):
                    r = g * S + s
                    rows[r, pl.ds(off, L)] = rows[r, pl.ds(off, L)] + ps[s]
            return c
        lax.fori_loop(0, H // L, jbody, 0)

    def fire_gather(k, b):
        return pltpu.async_copy(tok_hbm.at[idx_v.at[k]], bufs[b], gsem[b])

    def wait_gather(k, b):
        pltpu.make_async_copy(tok_hbm.at[idx_v.at[k]], bufs[b], gsem[b]).wait()

    def fire_scatter(k, b):
        return pltpu.async_copy(
            bufs[b], out_hbm.at[pl.ds(base + k * C, C)], ssem[b])

    def wait_scatter(k, b):
        pltpu.make_async_copy(
            bufs[b], out_hbm.at[pl.ds(base + k * C, C)], ssem[b]).wait()

    def step(k, b, swait, gfire):
        wait_gather(k, b)
        bn = (b + G) % NBUF
        if swait:
            wait_scatter(k + G - NBUF, bn)
        if gfire:
            fire_gather(k + G, bn)
        add_pos(bufs[b])
        fire_scatter(k, b)

    for j in range(G):
        fire_gather(j, j)
    head = NBUF - G
    for k in range(head):
        step(k, k, swait=False, gfire=True)

    n_main = (NCHUNK - head - G) // NBUF

    def main_wrap(kq, c):
        k0 = head + kq * NBUF
        for j in range(NBUF):
            k = k0 + j
            b = (head + j) % NBUF
            step(k, b, swait=True, gfire=True)
        return c

    lax.fori_loop(0, n_main, main_wrap, 0)

    for k in range(head + n_main * NBUF, NCHUNK - G):
        step(k, k % NBUF, swait=True, gfire=True)

    for j in range(G):
        k = NCHUNK - G + j
        b = k % NBUF
        step(k, b, swait=True, gfire=False)

    for j in range(NBUF - G):
        k = NCHUNK - (NBUF - G) + j
        wait_scatter(k, k % NBUF)


def kernel(input_ids, token_embedding, position_embedding):
    ids = input_ids.astype(jnp.int32).reshape(NW, NCHUNK, C)
    out = _emb(ids, token_embedding, position_embedding)
    return out.reshape(BATCH, S, H)


# speedup vs baseline: 1.4277x; 1.4277x over previous
import functools

import jax
import jax.numpy as jnp
from jax import lax
from jax.experimental import pallas as pl
from jax.experimental.pallas import tpu as pltpu
from jax.experimental.pallas import tpu_sc as plsc

VOCAB = 49408
H = 512
S = 16
BATCH = 4096
N = BATCH * S
L = 16
NC, NS = 2, 16
NW = NC * NS
BPW = N // NW
C = 32
NCHUNK = BPW // C
NBUF = 6
G = 3

_mesh = plsc.VectorSubcoreMesh(core_axis_name="c", subcore_axis_name="s")


@functools.partial(
    pl.kernel,
    out_type=jax.ShapeDtypeStruct((N, H), jnp.float32),
    mesh=_mesh,
    scratch_types=[
        pltpu.VMEM((NCHUNK, C), jnp.int32),
        pltpu.VMEM((S, H), jnp.float32),
    ] + [pltpu.VMEM((C, H), jnp.float32) for _ in range(NBUF)]
      + [pltpu.SemaphoreType.DMA for _ in range(2 * NBUF)],
)
def _emb(ids_hbm, tok_hbm, pos_hbm, out_hbm, idx_v, pos_v, *bs):
    bufs = bs[:NBUF]
    gsem = bs[NBUF:2 * NBUF]
    ssem = bs[2 * NBUF:]
    wid = lax.axis_index("s") * NC + lax.axis_index("c")
    base = wid * BPW
    pltpu.sync_copy(ids_hbm.at[wid], idx_v)
    pltpu.sync_copy(pos_hbm, pos_v)

    def add_pos(rows):
        def jbody(j, c):
            off = j * L
            ps = [pos_v[s, pl.ds(off, L)] for s in range(S)]
            for g in range(C // S):
                for s in range(S):
                    r = g * S + s
                    rows[r, pl.ds(off, L)] = rows[r, pl.ds(off, L)] + ps[s]
            return c
        lax.fori_loop(0, H // L, jbody, 0)

    def fire_gather(k, b):
        return pltpu.async_copy(tok_hbm.at[idx_v.at[k]], bufs[b], gsem[b])

    def wait_gather(k, b):
        pltpu.make_async_copy(tok_hbm.at[idx_v.at[k]], bufs[b], gsem[b]).wait()

    def fire_scatter(k, b):
        return pltpu.async_copy(
            bufs[b], out_hbm.at[pl.ds(base + k * C, C)], ssem[b])

    def wait_scatter(k, b):
        pltpu.make_async_copy(
            bufs[b], out_hbm.at[pl.ds(base + k * C, C)], ssem[b]).wait()

    def step(k, b, swait, gfire):
        wait_gather(k, b)
        bn = (b + G) % NBUF
        if swait:
            wait_scatter(k + G - NBUF, bn)
        if gfire:
            fire_gather(k + G, bn)
        add_pos(bufs[b])
        fire_scatter(k, b)

    for j in range(G):
        fire_gather(j, j)
    head = NBUF - G
    for k in range(head):
        step(k, k, swait=False, gfire=True)

    n_main = (NCHUNK - head - G) // NBUF

    def main_wrap(kq, c):
        k0 = head + kq * NBUF
        for j in range(NBUF):
            k = k0 + j
            b = (head + j) % NBUF
            step(k, b, swait=True, gfire=True)
        return c

    lax.fori_loop(0, n_main, main_wrap, 0)

    for k in range(head + n_main * NBUF, NCHUNK - G):
        step(k, k % NBUF, swait=True, gfire=True)

    for j in range(G):
        k = NCHUNK - G + j
        b = k % NBUF
        step(k, b, swait=True, gfire=False)

    for j in range(NBUF - G):
        k = NCHUNK - (NBUF - G) + j
        wait_scatter(k, k % NBUF)


def kernel(input_ids, token_embedding, position_embedding):
    ids = input_ids.astype(jnp.int32).reshape(NW, NCHUNK, C)
    out = _emb(ids, token_embedding, position_embedding)
    return out.reshape(BATCH, S, H)


# speedup vs baseline: 1.6870x; 1.1816x over previous
import functools

import jax
import jax.numpy as jnp
from jax import lax
from jax.experimental import pallas as pl
from jax.experimental.pallas import tpu as pltpu
from jax.experimental.pallas import tpu_sc as plsc

VOCAB = 49408
H = 512
S = 16
BATCH = 4096
N = BATCH * S
L = 16
NC, NS = 2, 16
NW = NC * NS
BPW = N // NW
C = 32
NCHUNK = BPW // C
NBUF = 6
G = 2

_mesh = plsc.VectorSubcoreMesh(core_axis_name="c", subcore_axis_name="s")


@functools.partial(
    pl.kernel,
    out_type=jax.ShapeDtypeStruct((N, H), jnp.float32),
    mesh=_mesh,
    scratch_types=[
        pltpu.VMEM((NCHUNK, C), jnp.int32),
        pltpu.VMEM((S, H), jnp.float32),
        pltpu.VMEM((NBUF * C, H), jnp.float32),
    ] + [pltpu.SemaphoreType.DMA for _ in range(NBUF)]
      + [pltpu.SemaphoreType.DMA for _ in range(NBUF // 2)],
)
def _emb(ids_hbm, tok_hbm, pos_hbm, out_hbm, idx_v, pos_v, ring, *sems):
    gsem = sems[:NBUF]
    ssem = sems[NBUF:]
    wid = lax.axis_index("s") * NC + lax.axis_index("c")
    base = wid * BPW
    pltpu.sync_copy(ids_hbm.at[wid], idx_v)
    pltpu.sync_copy(pos_hbm, pos_v)

    def add_pos(b):
        def jbody(j, c):
            off = j * L
            ps = [pos_v[s, pl.ds(off, L)] for s in range(S)]
            for g in range(C // S):
                for s in range(S):
                    r = b * C + g * S + s
                    ring[r, pl.ds(off, L)] = ring[r, pl.ds(off, L)] + ps[s]
            return c
        lax.fori_loop(0, H // L, jbody, 0)

    def fire_gather(k, b):
        return pltpu.async_copy(
            tok_hbm.at[idx_v.at[k]], ring.at[pl.ds(b * C, C)], gsem[b])

    def wait_gather(k, b):
        pltpu.make_async_copy(
            tok_hbm.at[idx_v.at[k]], ring.at[pl.ds(b * C, C)], gsem[b]).wait()

    def fire_scatter(k, b):
        return pltpu.async_copy(
            ring.at[pl.ds((b - 1) * C, 2 * C)],
            out_hbm.at[pl.ds(base + (k - 1) * C, 2 * C)],
            ssem[(b - 1) // 2])

    def wait_scatter(k, b):
        pltpu.make_async_copy(
            ring.at[pl.ds((b - 1) * C, 2 * C)],
            out_hbm.at[pl.ds(base + (k - 1) * C, 2 * C)],
            ssem[(b - 1) // 2]).wait()

    def step(k, b, swait, gfire):
        wait_gather(k, b)
        bn = (b + G) % NBUF
        if gfire:
            if swait and bn % 2 == 0:
                wait_scatter(k + G - NBUF + 1, bn + 1)
            fire_gather(k + G, bn)
        add_pos(b)
        if b % 2 == 1:
            fire_scatter(k, b)

    for j in range(G):
        fire_gather(j, j)
    head = NBUF - G
    for k in range(head):
        step(k, k, swait=False, gfire=True)

    n_main = (NCHUNK - head - G) // NBUF

    def main_wrap(kq, c):
        k0 = head + kq * NBUF
        for j in range(NBUF):
            k = k0 + j
            b = (head + j) % NBUF
            step(k, b, swait=True, gfire=True)
        return c

    lax.fori_loop(0, n_main, main_wrap, 0)

    for k in range(head + n_main * NBUF, NCHUNK - G):
        step(k, k % NBUF, swait=True, gfire=True)

    for j in range(G):
        k = NCHUNK - G + j
        step(k, k % NBUF, swait=True, gfire=False)

    for k in (NCHUNK - 5, NCHUNK - 3, NCHUNK - 1):
        wait_scatter(k, k % NBUF)


def kernel(input_ids, token_embedding, position_embedding):
    ids = input_ids.astype(jnp.int32).reshape(NW, NCHUNK, C)
    out = _emb(ids, token_embedding, position_embedding)
    return out.reshape(BATCH, S, H)
